# baseline (speedup 1.0000x reference)
import jax
import jax.numpy as jnp
from jax.experimental import pallas as pl
from jax.experimental.pallas import tpu as pltpu

R = 512
N = 16
K = 8
PE_SLOTS = 4


def kernel(x, pos_embedding):
    B, T, D = x.shape
    NT = T // R
    nchunks = B * NT
    x2 = x.reshape(B * T, D)
    pe = pos_embedding[:T]

    def body(x_ref, pe_ref, o_ref, buf, pebuf, insem, outsem, pesem):
        def base(i):
            tc, b = divmod(i, B)
            return b * T + tc * R

        def in_cp(i):
            s = i % N
            return pltpu.make_async_copy(
                x_ref.at[pl.ds(base(i), R)], buf.at[s], insem.at[s])

        def out_cp(i):
            s = i % N
            return pltpu.make_async_copy(
                buf.at[s], o_ref.at[pl.ds(base(i), R)], outsem.at[s])

        def pe_cp(tc):
            s = tc % PE_SLOTS
            return pltpu.make_async_copy(
                pe_ref.at[pl.ds(tc * R, R)], pebuf.at[s], pesem.at[s])

        for tc in range(min(PE_SLOTS - 1, NT)):
            pe_cp(tc).start()
        for i in range(min(K, nchunks)):
            in_cp(i).start()

        for i in range(nchunks):
            s = i % N
            tc, b = divmod(i, B)
            if b == 0:
                pe_cp(tc).wait()
                if tc + PE_SLOTS - 1 < NT:
                    pe_cp(tc + PE_SLOTS - 1).start()
            in_cp(i).wait()
            buf[s] = buf[s] + pebuf[tc % PE_SLOTS]
            out_cp(i).start()
            j = i + K
            if j < nchunks:
                if j >= N:
                    out_cp(j - N).wait()
                in_cp(j).start()

        for c in range(max(0, nchunks - N), nchunks):
            out_cp(c).wait()

    out = pl.pallas_call(
        body,
        in_specs=[
            pl.BlockSpec(memory_space=pltpu.MemorySpace.HBM),
            pl.BlockSpec(memory_space=pltpu.MemorySpace.HBM),
        ],
        out_specs=pl.BlockSpec(memory_space=pltpu.MemorySpace.HBM),
        out_shape=jax.ShapeDtypeStruct((B * T, D), x.dtype),
        scratch_shapes=[
            pltpu.VMEM((N, R, D), x.dtype),
            pltpu.VMEM((PE_SLOTS, R, D), x.dtype),
            pltpu.SemaphoreType.DMA((N,)),
            pltpu.SemaphoreType.DMA((N,)),
            pltpu.SemaphoreType.DMA((PE_SLOTS,)),
        ],
    )(x2, pe)
    return out.reshape(B, T, D)


# speedup vs baseline: 1.0029x; 1.0029x over previous
import jax
import jax.numpy as jnp
from jax.experimental import pallas as pl


def _add_kernel(x_ref, pe_ref, o_ref):
    o_ref[...] = x_ref[...] + pe_ref[...]


def kernel(x, pos_embedding):
    B, T, D = x.shape
    pe = pos_embedding[:T]
    BT = 2048
    grid = (T // BT, B)
    return pl.pallas_call(
        _add_kernel,
        grid=grid,
        in_specs=[
            pl.BlockSpec((1, BT, D), lambda t, b: (b, t, 0)),
            pl.BlockSpec((BT, D), lambda t, b: (t, 0)),
        ],
        out_specs=pl.BlockSpec((1, BT, D), lambda t, b: (b, t, 0)),
        out_shape=jax.ShapeDtypeStruct((B, T, D), x.dtype),
    )(x, pe)
